# baseline (speedup 1.0000x reference)
import functools

import jax
import jax.numpy as jnp
from jax.experimental import pallas as pl
from jax.experimental.pallas import tpu as pltpu

NH = 12
CAP_FRAC = 1.25
_QCHUNK = 512


def _transpose_col(v, eye):
    return jax.lax.dot_general(v, eye, (((0,), (0,)), ((), ())),
                               preferred_element_type=jnp.float32,
                               precision=jax.lax.Precision.HIGHEST)


def _routing_kernel(x_ref, wsc_ref, bsc_ref, wsu_ref, bsu_ref,
                    rc_ref, ru_ref, dd_ref, acc_ref, *, cap):
    b = pl.program_id(0)
    nb = pl.num_programs(0)
    seq = x_ref.shape[1]
    xm = jnp.sum(x_ref[0], axis=0, keepdims=True) * (1.0 / seq)
    acc_ref[pl.ds(b, 1), :] = xm

    @pl.when(b == nb - 1)
    def _finalize():
        xall = acc_ref[...]
        bsz = xall.shape[0]
        eye = (jax.lax.broadcasted_iota(jnp.int32, (bsz, bsz), 0) ==
               jax.lax.broadcasted_iota(jnp.int32, (bsz, bsz), 1)
               ).astype(jnp.float32)

        def route(w_ref, b_ref):
            logits = jnp.dot(xall, w_ref[...],
                             preferred_element_type=jnp.float32,
                             precision=jax.lax.Precision.HIGHEST) + b_ref[...]
            p = jax.nn.softmax(logits, axis=-1)
            pmax = jnp.max(p, axis=-1, keepdims=True)
            ne = logits.shape[1]
            col = jax.lax.broadcasted_iota(jnp.int32, logits.shape, 1)
            r = jnp.min(jnp.where(p >= pmax, col, ne), axis=-1,
                        keepdims=True)
            return r.astype(jnp.float32), pmax

        rc, pmc = route(wsc_ref, bsc_ref)
        ru, _ = route(wsu_ref, bsu_ref)

        rc_row = _transpose_col(rc, eye)
        pm_row = _transpose_col(pmc, eye)
        idx_col = jax.lax.broadcasted_iota(jnp.int32, (bsz, bsz), 0)
        idx_row = jax.lax.broadcasted_iota(jnp.int32, (bsz, bsz), 1)
        same = rc_row == rc
        beats = (pm_row > pmc) | ((pm_row == pmc) & (idx_row < idx_col))
        rank = jnp.sum(jnp.where(same & beats, 1.0, 0.0), axis=-1,
                       keepdims=True)
        dropped = jnp.where(rank >= cap, 1.0, 0.0)

        rc_ref[...] = rc_row.astype(jnp.int32)
        ru_ref[...] = _transpose_col(ru, eye).astype(jnp.int32)
        dd_ref[...] = _transpose_col(dropped, eye).astype(jnp.int32)


def _expert_kernel(rc_ref, ru_ref, dd_ref,
                   x_ref, mask_ref,
                   wq, bq, wk, bk, wv, bv, wo, bo,
                   o_ref, ctx_ref, q_ref, k_ref, v_ref):
    b = pl.program_id(0)
    j = pl.program_id(1)
    x = x_ref[0]
    seq, hid = x.shape
    dh = hid // NH
    mrow = mask_ref[0].astype(jnp.float32)

    @pl.when((j == 0) | (dd_ref[b] == 1))
    def _compute():
        q_ref[...] = jnp.dot(x, wq[0],
                             preferred_element_type=jnp.float32) + bq[0]
        k_ref[...] = jnp.dot(x, wk[0],
                             preferred_element_type=jnp.float32) + bk[0]
        v_ref[...] = jnp.dot(x, wv[0],
                             preferred_element_type=jnp.float32) + bv[0]
        for h in range(NH):
            sl = slice(h * dh, (h + 1) * dh)
            kh, vh = k_ref[:, sl], v_ref[:, sl]
            for c0 in range(0, seq, _QCHUNK):
                rs = slice(c0, c0 + _QCHUNK)
                s = jax.lax.dot_general(q_ref[rs, sl], kh,
                                        (((1,), (1,)), ((), ())),
                                        preferred_element_type=jnp.float32)
                m = jnp.max(s, axis=-1, keepdims=True)
                e = jnp.exp(s - m) * mrow
                r = 1.0 / jnp.sum(e, axis=-1, keepdims=True)
                ctx_ref[rs, sl] = jnp.dot(
                    e, vh, preferred_element_type=jnp.float32) * r
        res = jnp.dot(ctx_ref[...], wo[0],
                      preferred_element_type=jnp.float32) + bo[0]
        o_ref[0] = jnp.where(j == 0, res, res + o_ref[0])


def kernel(hidden_states, attention_mask, Wsc, bsc, Wsu, bsu,
           cWq, cbq, cWk, cbk, cWv, cbv, cWo, cbo,
           uWq, ubq, uWk, ubk, uWv, ubv, uWo, ubo):
    x = hidden_states
    B, S, H = x.shape
    EC = Wsc.shape[1]
    EU = Wsu.shape[1]
    cap = int(CAP_FRAC * B / EC)

    rc, ru, dd = pl.pallas_call(
        functools.partial(_routing_kernel, cap=cap),
        grid=(B,),
        in_specs=[
            pl.BlockSpec((1, S, H), lambda b: (b, 0, 0)),
            pl.BlockSpec((H, EC), lambda b: (0, 0)),
            pl.BlockSpec((1, EC), lambda b: (0, 0)),
            pl.BlockSpec((H, EU), lambda b: (0, 0)),
            pl.BlockSpec((1, EU), lambda b: (0, 0)),
        ],
        out_specs=[
            pl.BlockSpec((1, B), lambda b: (0, 0)),
            pl.BlockSpec((1, B), lambda b: (0, 0)),
            pl.BlockSpec((1, B), lambda b: (0, 0)),
        ],
        out_shape=[
            jax.ShapeDtypeStruct((1, B), jnp.int32),
            jax.ShapeDtypeStruct((1, B), jnp.int32),
            jax.ShapeDtypeStruct((1, B), jnp.int32),
        ],
        scratch_shapes=[pltpu.VMEM((B, H), jnp.float32)],
    )(x, Wsc, bsc.reshape(1, EC), Wsu, bsu.reshape(1, EU))
    rc = rc.reshape(B)
    ru = ru.reshape(B)
    dd = dd.reshape(B)

    mask2 = attention_mask.reshape(B, 1, S)

    def expert_idx(b, j, rcs, rus, dds):
        e = jnp.where(j == 0, rcs[b],
                      jnp.where(dds[b] == 1, rus[b] + EC, rcs[b]))
        return (e, 0, 0)

    wspec = pl.BlockSpec((1, H, H), expert_idx)
    bspec = pl.BlockSpec((1, 1, H), expert_idx)

    grid_spec = pltpu.PrefetchScalarGridSpec(
        num_scalar_prefetch=3,
        grid=(B, 2),
        in_specs=[
            pl.BlockSpec((1, S, H), lambda b, j, rcs, rus, dds: (b, 0, 0)),
            pl.BlockSpec((1, 1, S), lambda b, j, rcs, rus, dds: (b, 0, 0)),
            wspec, bspec, wspec, bspec, wspec, bspec, wspec, bspec,
        ],
        out_specs=pl.BlockSpec((1, S, H),
                               lambda b, j, rcs, rus, dds: (b, 0, 0)),
        scratch_shapes=[pltpu.VMEM((S, H), jnp.float32)] * 4,
    )

    scale = 1.0 / ((H // NH) ** 0.5)
    cat = jnp.concatenate
    out = pl.pallas_call(
        _expert_kernel,
        grid_spec=grid_spec,
        out_shape=jax.ShapeDtypeStruct((B, S, H), jnp.float32),
    )(rc, ru, dd, x, mask2,
      cat([cWq, uWq]) * scale,
      cat([cbq, ubq]).reshape(EC + EU, 1, H) * scale,
      cat([cWk, uWk]), cat([cbk, ubk]).reshape(EC + EU, 1, H),
      cat([cWv, uWv]), cat([cbv, ubv]).reshape(EC + EU, 1, H),
      cat([cWo, uWo]), cat([cbo, ubo]).reshape(EC + EU, 1, H))
    return out


# speedup vs baseline: 1.6403x; 1.6403x over previous
import functools

import jax
import jax.numpy as jnp
from jax.experimental import pallas as pl
from jax.experimental.pallas import tpu as pltpu

NH = 12
CAP_FRAC = 1.25
_QCHUNK = 256


def _transpose_col(v, eye):
    return jax.lax.dot_general(v, eye, (((0,), (0,)), ((), ())),
                               preferred_element_type=jnp.float32,
                               precision=jax.lax.Precision.HIGHEST)


def _routing_kernel(x_ref, wsc_ref, bsc_ref, wsu_ref, bsu_ref,
                    rc_ref, ru_ref, dd_ref, acc_ref, *, cap):
    b = pl.program_id(0)
    nb = pl.num_programs(0)
    seq = x_ref.shape[1]
    xm = jnp.sum(x_ref[0], axis=0, keepdims=True) * (1.0 / seq)
    acc_ref[pl.ds(b, 1), :] = xm

    @pl.when(b == nb - 1)
    def _finalize():
        xall = acc_ref[...]
        bsz = xall.shape[0]
        eye = (jax.lax.broadcasted_iota(jnp.int32, (bsz, bsz), 0) ==
               jax.lax.broadcasted_iota(jnp.int32, (bsz, bsz), 1)
               ).astype(jnp.float32)

        def route(w_ref, b_ref):
            logits = jnp.dot(xall, w_ref[...],
                             preferred_element_type=jnp.float32,
                             precision=jax.lax.Precision.HIGHEST) + b_ref[...]
            p = jax.nn.softmax(logits, axis=-1)
            pmax = jnp.max(p, axis=-1, keepdims=True)
            ne = logits.shape[1]
            col = jax.lax.broadcasted_iota(jnp.int32, logits.shape, 1)
            r = jnp.min(jnp.where(p >= pmax, col, ne), axis=-1,
                        keepdims=True)
            return r.astype(jnp.float32), pmax

        rc, pmc = route(wsc_ref, bsc_ref)
        ru, _ = route(wsu_ref, bsu_ref)

        rc_row = _transpose_col(rc, eye)
        pm_row = _transpose_col(pmc, eye)
        idx_col = jax.lax.broadcasted_iota(jnp.int32, (bsz, bsz), 0)
        idx_row = jax.lax.broadcasted_iota(jnp.int32, (bsz, bsz), 1)
        same = rc_row == rc
        beats = (pm_row > pmc) | ((pm_row == pmc) & (idx_row < idx_col))
        rank = jnp.sum(jnp.where(same & beats, 1.0, 0.0), axis=-1,
                       keepdims=True)
        dropped = jnp.where(rank >= cap, 1.0, 0.0)

        rc_ref[...] = rc_row.astype(jnp.int32)
        ru_ref[...] = _transpose_col(ru, eye).astype(jnp.int32)
        dd_ref[...] = _transpose_col(dropped, eye).astype(jnp.int32)


def _expert_kernel(rc_ref, ru_ref, dd_ref,
                   x_ref, mask_ref,
                   cwq, cbq, cwk, cbk, cwv, cbv, cwo, cbo,
                   uwq, ubq, uwk, ubk, uwv, ubv, uwo, ubo,
                   o_ref, ctx_ref, q_ref, k_ref, v_ref):
    b = pl.program_id(0)
    x = x_ref[0]
    seq, hid = x.shape
    dh = hid // NH
    mrow = mask_ref[0].astype(jnp.float32)

    def mha(wq, bq, wk, bk, wv, bv, wo, bo):
        q_ref[...] = jnp.dot(x, wq[0],
                             preferred_element_type=jnp.float32) + bq[0]
        k_ref[...] = jnp.dot(x, wk[0],
                             preferred_element_type=jnp.float32) + bk[0]
        v_ref[...] = jnp.dot(x, wv[0],
                             preferred_element_type=jnp.float32) + bv[0]
        for h in range(NH):
            sl = slice(h * dh, (h + 1) * dh)
            kh, vh = k_ref[:, sl], v_ref[:, sl]
            for c0 in range(0, seq, _QCHUNK):
                rs = slice(c0, c0 + _QCHUNK)
                s = jax.lax.dot_general(q_ref[rs, sl], kh,
                                        (((1,), (1,)), ((), ())),
                                        preferred_element_type=jnp.float32)
                m = jnp.max(s, axis=-1, keepdims=True)
                e = jnp.exp(s - m) * mrow
                r = 1.0 / jnp.sum(e, axis=-1, keepdims=True)
                ctx_ref[rs, sl] = jnp.dot(
                    e, vh, preferred_element_type=jnp.float32) * r
        return jnp.dot(ctx_ref[...].astype(jnp.bfloat16), wo[0],
                       preferred_element_type=jnp.float32) + bo[0]

    o_ref[0] = mha(cwq, cbq, cwk, cbk, cwv, cbv, cwo, cbo)

    @pl.when(dd_ref[b] == 1)
    def _tail():
        o_ref[0] = o_ref[0] + mha(uwq, ubq, uwk, ubk, uwv, ubv, uwo, ubo)


def kernel(hidden_states, attention_mask, Wsc, bsc, Wsu, bsu,
           cWq, cbq, cWk, cbk, cWv, cbv, cWo, cbo,
           uWq, ubq, uWk, ubk, uWv, ubv, uWo, ubo):
    x = hidden_states
    B, S, H = x.shape
    EC = Wsc.shape[1]
    EU = Wsu.shape[1]
    cap = int(CAP_FRAC * B / EC)

    rc, ru, dd = pl.pallas_call(
        functools.partial(_routing_kernel, cap=cap),
        grid=(B,),
        in_specs=[
            pl.BlockSpec((1, S, H), lambda b: (b, 0, 0)),
            pl.BlockSpec((H, EC), lambda b: (0, 0)),
            pl.BlockSpec((1, EC), lambda b: (0, 0)),
            pl.BlockSpec((H, EU), lambda b: (0, 0)),
            pl.BlockSpec((1, EU), lambda b: (0, 0)),
        ],
        out_specs=[
            pl.BlockSpec((1, B), lambda b: (0, 0)),
            pl.BlockSpec((1, B), lambda b: (0, 0)),
            pl.BlockSpec((1, B), lambda b: (0, 0)),
        ],
        out_shape=[
            jax.ShapeDtypeStruct((1, B), jnp.int32),
            jax.ShapeDtypeStruct((1, B), jnp.int32),
            jax.ShapeDtypeStruct((1, B), jnp.int32),
        ],
        scratch_shapes=[pltpu.VMEM((B, H), jnp.float32)],
    )(x, Wsc, bsc.reshape(1, EC), Wsu, bsu.reshape(1, EU))
    rc = rc.reshape(B)
    ru = ru.reshape(B)
    dd = dd.reshape(B)

    mask2 = attention_mask.reshape(B, 1, S)

    def wspec():
        return pl.BlockSpec((1, H, H),
                            lambda b, rcs, rus, dds: (rcs[b], 0, 0))

    def bspec():
        return pl.BlockSpec((1, 1, H),
                            lambda b, rcs, rus, dds: (rcs[b], 0, 0))

    def uwspec():
        return pl.BlockSpec(
            (1, H, H),
            lambda b, rcs, rus, dds: (jnp.where(dds[b] == 1, rus[b], 0), 0, 0))

    def ubspec():
        return pl.BlockSpec(
            (1, 1, H),
            lambda b, rcs, rus, dds: (jnp.where(dds[b] == 1, rus[b], 0), 0, 0))

    grid_spec = pltpu.PrefetchScalarGridSpec(
        num_scalar_prefetch=3,
        grid=(B,),
        in_specs=[
            pl.BlockSpec((1, S, H), lambda b, rcs, rus, dds: (b, 0, 0)),
            pl.BlockSpec((1, 1, S), lambda b, rcs, rus, dds: (b, 0, 0)),
            wspec(), bspec(), wspec(), bspec(),
            wspec(), bspec(), wspec(), bspec(),
            uwspec(), ubspec(), uwspec(), ubspec(),
            uwspec(), ubspec(), uwspec(), ubspec(),
        ],
        out_specs=pl.BlockSpec((1, S, H), lambda b, rcs, rus, dds: (b, 0, 0)),
        scratch_shapes=[pltpu.VMEM((S, H), jnp.float32)] * 4,
    )

    scale = 1.0 / ((H // NH) ** 0.5)
    bf = jnp.bfloat16
    out = pl.pallas_call(
        _expert_kernel,
        grid_spec=grid_spec,
        out_shape=jax.ShapeDtypeStruct((B, S, H), jnp.float32),
    )(rc, ru, dd, x.astype(bf), mask2,
      (cWq * scale).astype(bf), cbq.reshape(EC, 1, H) * scale,
      cWk.astype(bf), cbk.reshape(EC, 1, H),
      cWv.astype(bf), cbv.reshape(EC, 1, H),
      cWo.astype(bf), cbo.reshape(EC, 1, H),
      (uWq * scale).astype(bf), ubq.reshape(EU, 1, H) * scale,
      uWk.astype(bf), ubk.reshape(EU, 1, H),
      uWv.astype(bf), ubv.reshape(EU, 1, H),
      uWo.astype(bf), ubo.reshape(EU, 1, H))
    return out


# speedup vs baseline: 1.6556x; 1.0093x over previous
import functools

import jax
import jax.numpy as jnp
from jax.experimental import pallas as pl
from jax.experimental.pallas import tpu as pltpu

NH = 12
CAP_FRAC = 1.25


def _transpose_col(v, eye):
    return jax.lax.dot_general(v, eye, (((0,), (0,)), ((), ())),
                               preferred_element_type=jnp.float32,
                               precision=jax.lax.Precision.HIGHEST)


def _routing_kernel(x_ref, wsc_ref, bsc_ref, wsu_ref, bsu_ref,
                    rc_ref, ru_ref, dd_ref, acc_ref, *, cap):
    b = pl.program_id(0)
    nb = pl.num_programs(0)
    seq = x_ref.shape[1]
    xm = jnp.sum(x_ref[0], axis=0, keepdims=True) * (1.0 / seq)
    acc_ref[pl.ds(b, 1), :] = xm

    @pl.when(b == nb - 1)
    def _finalize():
        xall = acc_ref[...]
        bsz = xall.shape[0]
        eye = (jax.lax.broadcasted_iota(jnp.int32, (bsz, bsz), 0) ==
               jax.lax.broadcasted_iota(jnp.int32, (bsz, bsz), 1)
               ).astype(jnp.float32)

        def route(w_ref, b_ref):
            logits = jnp.dot(xall, w_ref[...],
                             preferred_element_type=jnp.float32,
                             precision=jax.lax.Precision.HIGHEST) + b_ref[...]
            p = jax.nn.softmax(logits, axis=-1)
            pmax = jnp.max(p, axis=-1, keepdims=True)
            ne = logits.shape[1]
            col = jax.lax.broadcasted_iota(jnp.int32, logits.shape, 1)
            r = jnp.min(jnp.where(p >= pmax, col, ne), axis=-1,
                        keepdims=True)
            return r.astype(jnp.float32), pmax

        rc, pmc = route(wsc_ref, bsc_ref)
        ru, _ = route(wsu_ref, bsu_ref)

        rc_row = _transpose_col(rc, eye)
        pm_row = _transpose_col(pmc, eye)
        idx_col = jax.lax.broadcasted_iota(jnp.int32, (bsz, bsz), 0)
        idx_row = jax.lax.broadcasted_iota(jnp.int32, (bsz, bsz), 1)
        same = rc_row == rc
        beats = (pm_row > pmc) | ((pm_row == pmc) & (idx_row < idx_col))
        rank = jnp.sum(jnp.where(same & beats, 1.0, 0.0), axis=-1,
                       keepdims=True)
        dropped = jnp.where(rank >= cap, 1.0, 0.0)

        rc_ref[...] = rc_row.astype(jnp.int32)
        ru_ref[...] = _transpose_col(ru, eye).astype(jnp.int32)
        dd_ref[...] = _transpose_col(dropped, eye).astype(jnp.int32)


def _expert_kernel(rc_ref, ru_ref, dd_ref,
                   x_ref, mask_ref,
                   cwq, cbq, cwk, cbk, cwv, cbv, cwo, cbo,
                   uwq, ubq, uwk, ubk, uwv, ubv, uwo, ubo,
                   o_ref, ctx_ref):
    b = pl.program_id(0)
    x = x_ref[0]
    seq, hid = x.shape
    dh = hid // NH
    ext = (1.0 - mask_ref[0].astype(jnp.float32)) * -10000.0

    def mha(wq, bq, wk, bk, wv, bv, wo, bo):
        q = jnp.dot(x, wq[0], preferred_element_type=jnp.float32) + bq[0]
        k = jnp.dot(x, wk[0], preferred_element_type=jnp.float32) + bk[0]
        v = jnp.dot(x, wv[0], preferred_element_type=jnp.float32) + bv[0]
        for h in range(NH):
            sl = slice(h * dh, (h + 1) * dh)
            qh, kh, vh = q[:, sl], k[:, sl], v[:, sl]
            s = jax.lax.dot_general(qh, kh, (((1,), (1,)), ((), ())),
                                    preferred_element_type=jnp.float32)
            s = s + ext
            s = s - jnp.max(s, axis=-1, keepdims=True)
            e = jnp.exp(s)
            p = e / jnp.sum(e, axis=-1, keepdims=True)
            ctx_ref[:, sl] = jnp.dot(p, vh, preferred_element_type=jnp.float32)
        return jnp.dot(ctx_ref[...], wo[0],
                       preferred_element_type=jnp.float32) + bo[0]

    o_ref[0] = mha(cwq, cbq, cwk, cbk, cwv, cbv, cwo, cbo)

    @pl.when(dd_ref[b] == 1)
    def _tail():
        o_ref[0] = o_ref[0] + mha(uwq, ubq, uwk, ubk, uwv, ubv, uwo, ubo)


def kernel(hidden_states, attention_mask, Wsc, bsc, Wsu, bsu,
           cWq, cbq, cWk, cbk, cWv, cbv, cWo, cbo,
           uWq, ubq, uWk, ubk, uWv, ubv, uWo, ubo):
    x = hidden_states
    B, S, H = x.shape
    EC = Wsc.shape[1]
    EU = Wsu.shape[1]
    cap = int(CAP_FRAC * B / EC)

    rc, ru, dd = pl.pallas_call(
        functools.partial(_routing_kernel, cap=cap),
        grid=(B,),
        in_specs=[
            pl.BlockSpec((1, S, H), lambda b: (b, 0, 0)),
            pl.BlockSpec((H, EC), lambda b: (0, 0)),
            pl.BlockSpec((1, EC), lambda b: (0, 0)),
            pl.BlockSpec((H, EU), lambda b: (0, 0)),
            pl.BlockSpec((1, EU), lambda b: (0, 0)),
        ],
        out_specs=[
            pl.BlockSpec((1, B), lambda b: (0, 0)),
            pl.BlockSpec((1, B), lambda b: (0, 0)),
            pl.BlockSpec((1, B), lambda b: (0, 0)),
        ],
        out_shape=[
            jax.ShapeDtypeStruct((1, B), jnp.int32),
            jax.ShapeDtypeStruct((1, B), jnp.int32),
            jax.ShapeDtypeStruct((1, B), jnp.int32),
        ],
        scratch_shapes=[pltpu.VMEM((B, H), jnp.float32)],
    )(x, Wsc, bsc.reshape(1, EC), Wsu, bsu.reshape(1, EU))
    rc = rc.reshape(B)
    ru = ru.reshape(B)
    dd = dd.reshape(B)

    mask2 = attention_mask.reshape(B, 1, S)

    def wspec():
        return pl.BlockSpec((1, H, H),
                            lambda b, rcs, rus, dds: (rcs[b], 0, 0))

    def bspec():
        return pl.BlockSpec((1, 1, H),
                            lambda b, rcs, rus, dds: (rcs[b], 0, 0))

    def uwspec():
        return pl.BlockSpec(
            (1, H, H),
            lambda b, rcs, rus, dds: (jnp.where(dds[b] == 1, rus[b], 0), 0, 0))

    def ubspec():
        return pl.BlockSpec(
            (1, 1, H),
            lambda b, rcs, rus, dds: (jnp.where(dds[b] == 1, rus[b], 0), 0, 0))

    grid_spec = pltpu.PrefetchScalarGridSpec(
        num_scalar_prefetch=3,
        grid=(B,),
        in_specs=[
            pl.BlockSpec((1, S, H), lambda b, rcs, rus, dds: (b, 0, 0)),
            pl.BlockSpec((1, 1, S), lambda b, rcs, rus, dds: (b, 0, 0)),
            wspec(), bspec(), wspec(), bspec(),
            wspec(), bspec(), wspec(), bspec(),
            uwspec(), ubspec(), uwspec(), ubspec(),
            uwspec(), ubspec(), uwspec(), ubspec(),
        ],
        out_specs=pl.BlockSpec((1, S, H), lambda b, rcs, rus, dds: (b, 0, 0)),
        scratch_shapes=[pltpu.VMEM((S, H), jnp.float32)],
    )

    scale = 1.0 / ((H // NH) ** 0.5)
    out = pl.pallas_call(
        _expert_kernel,
        grid_spec=grid_spec,
        out_shape=jax.ShapeDtypeStruct((B, S, H), jnp.float32),
    )(rc, ru, dd, x, mask2,
      cWq * scale, cbq.reshape(EC, 1, H) * scale,
      cWk, cbk.reshape(EC, 1, H),
      cWv, cbv.reshape(EC, 1, H), cWo, cbo.reshape(EC, 1, H),
      uWq * scale, ubq.reshape(EU, 1, H) * scale,
      uWk, ubk.reshape(EU, 1, H),
      uWv, ubv.reshape(EU, 1, H), uWo, ubo.reshape(EU, 1, H))
    return out


# speedup vs baseline: 1.7470x; 1.0552x over previous
import functools

import jax
import jax.numpy as jnp
from jax.experimental import pallas as pl
from jax.experimental.pallas import tpu as pltpu

NH = 12
CAP_FRAC = 1.25


def _transpose_col(v, eye):
    return jax.lax.dot_general(v, eye, (((0,), (0,)), ((), ())),
                               preferred_element_type=jnp.float32,
                               precision=jax.lax.Precision.HIGHEST)


def _routing_kernel(x_ref, wsc_ref, bsc_ref, wsu_ref, bsu_ref,
                    rc_ref, ru_ref, dd_ref, acc_ref, *, cap):
    b = pl.program_id(0)
    nb = pl.num_programs(0)
    seq = x_ref.shape[1]
    xm = jnp.sum(x_ref[0], axis=0, keepdims=True) * (1.0 / seq)
    acc_ref[pl.ds(b, 1), :] = xm

    @pl.when(b == nb - 1)
    def _finalize():
        xall = acc_ref[...]
        bsz = xall.shape[0]
        eye = (jax.lax.broadcasted_iota(jnp.int32, (bsz, bsz), 0) ==
               jax.lax.broadcasted_iota(jnp.int32, (bsz, bsz), 1)
               ).astype(jnp.float32)

        def route(w_ref, b_ref):
            logits = jnp.dot(xall, w_ref[...],
                             preferred_element_type=jnp.float32,
                             precision=jax.lax.Precision.HIGHEST) + b_ref[...]
            p = jax.nn.softmax(logits, axis=-1)
            pmax = jnp.max(p, axis=-1, keepdims=True)
            ne = logits.shape[1]
            col = jax.lax.broadcasted_iota(jnp.int32, logits.shape, 1)
            r = jnp.min(jnp.where(p >= pmax, col, ne), axis=-1,
                        keepdims=True)
            return r.astype(jnp.float32), pmax

        rc, pmc = route(wsc_ref, bsc_ref)
        ru, _ = route(wsu_ref, bsu_ref)

        rc_row = _transpose_col(rc, eye)
        pm_row = _transpose_col(pmc, eye)
        idx_col = jax.lax.broadcasted_iota(jnp.int32, (bsz, bsz), 0)
        idx_row = jax.lax.broadcasted_iota(jnp.int32, (bsz, bsz), 1)
        same = rc_row == rc
        beats = (pm_row > pmc) | ((pm_row == pmc) & (idx_row < idx_col))
        rank = jnp.sum(jnp.where(same & beats, 1.0, 0.0), axis=-1,
                       keepdims=True)
        dropped = jnp.where(rank >= cap, 1.0, 0.0)

        rc_ref[...] = rc_row.astype(jnp.int32)
        ru_ref[...] = _transpose_col(ru, eye).astype(jnp.int32)
        dd_ref[...] = _transpose_col(dropped, eye).astype(jnp.int32)


def _expert_kernel(rc_ref, ru_ref, dd_ref,
                   x_ref, mask_ref,
                   cwq, cbq, cwk, cbk, cwv, cbv, cwo, cbo,
                   uwq, ubq, uwk, ubk, uwv, ubv, uwo, ubo,
                   o_ref, ctx_ref):
    b = pl.program_id(0)
    x = x_ref[0]
    seq, hid = x.shape
    dh = hid // NH
    scale = 1.0 / (dh ** 0.5)
    ext = (1.0 - mask_ref[0].astype(jnp.float32)) * -10000.0

    def mha(wq, bq, wk, bk, wv, bv, wo, bo):
        q = (jnp.dot(x, wq[0], preferred_element_type=jnp.float32)
             + bq[0]) * scale
        k = jnp.dot(x, wk[0], preferred_element_type=jnp.float32) + bk[0]
        v = jnp.dot(x, wv[0], preferred_element_type=jnp.float32) + bv[0]
        for h in range(NH):
            sl = slice(h * dh, (h + 1) * dh)
            qh, kh, vh = q[:, sl], k[:, sl], v[:, sl]
            s = jax.lax.dot_general(qh, kh, (((1,), (1,)), ((), ())),
                                    preferred_element_type=jnp.float32)
            s = s + ext
            s = s - jnp.max(s, axis=-1, keepdims=True)
            e = jnp.exp(s)
            p = e / jnp.sum(e, axis=-1, keepdims=True)
            ctx_ref[:, sl] = jnp.dot(p, vh, preferred_element_type=jnp.float32)
        return jnp.dot(ctx_ref[...], wo[0],
                       preferred_element_type=jnp.float32) + bo[0]

    o_ref[0] = mha(cwq, cbq, cwk, cbk, cwv, cbv, cwo, cbo)

    @pl.when(dd_ref[b] == 1)
    def _tail():
        o_ref[0] = o_ref[0] + mha(uwq, ubq, uwk, ubk, uwv, ubv, uwo, ubo)


def kernel(hidden_states, attention_mask, Wsc, bsc, Wsu, bsu,
           cWq, cbq, cWk, cbk, cWv, cbv, cWo, cbo,
           uWq, ubq, uWk, ubk, uWv, ubv, uWo, ubo):
    x = hidden_states
    B, S, H = x.shape
    EC = Wsc.shape[1]
    EU = Wsu.shape[1]
    cap = int(CAP_FRAC * B / EC)

    rc, ru, dd = pl.pallas_call(
        functools.partial(_routing_kernel, cap=cap),
        grid=(B,),
        in_specs=[
            pl.BlockSpec((1, S, H), lambda b: (b, 0, 0)),
            pl.BlockSpec((H, EC), lambda b: (0, 0)),
            pl.BlockSpec((1, EC), lambda b: (0, 0)),
            pl.BlockSpec((H, EU), lambda b: (0, 0)),
            pl.BlockSpec((1, EU), lambda b: (0, 0)),
        ],
        out_specs=[
            pl.BlockSpec((1, B), lambda b: (0, 0)),
            pl.BlockSpec((1, B), lambda b: (0, 0)),
            pl.BlockSpec((1, B), lambda b: (0, 0)),
        ],
        out_shape=[
            jax.ShapeDtypeStruct((1, B), jnp.int32),
            jax.ShapeDtypeStruct((1, B), jnp.int32),
            jax.ShapeDtypeStruct((1, B), jnp.int32),
        ],
        scratch_shapes=[pltpu.VMEM((B, H), jnp.float32)],
    )(x, Wsc, bsc.reshape(1, EC), Wsu, bsu.reshape(1, EU))
    rc = rc.reshape(B)
    ru = ru.reshape(B)
    dd = dd.reshape(B)

    mask2 = attention_mask.reshape(B, 1, S)

    def wspec():
        return pl.BlockSpec((1, H, H),
                            lambda b, rcs, rus, dds: (rcs[b], 0, 0))

    def bspec():
        return pl.BlockSpec((1, 1, H),
                            lambda b, rcs, rus, dds: (rcs[b], 0, 0))

    def uwspec():
        return pl.BlockSpec(
            (1, H, H),
            lambda b, rcs, rus, dds: (jnp.where(dds[b] == 1, rus[b], 0), 0, 0))

    def ubspec():
        return pl.BlockSpec(
            (1, 1, H),
            lambda b, rcs, rus, dds: (jnp.where(dds[b] == 1, rus[b], 0), 0, 0))

    grid_spec = pltpu.PrefetchScalarGridSpec(
        num_scalar_prefetch=3,
        grid=(B,),
        in_specs=[
            pl.BlockSpec((1, S, H), lambda b, rcs, rus, dds: (b, 0, 0)),
            pl.BlockSpec((1, 1, S), lambda b, rcs, rus, dds: (b, 0, 0)),
            wspec(), bspec(), wspec(), bspec(),
            wspec(), bspec(), wspec(), bspec(),
            uwspec(), ubspec(), uwspec(), ubspec(),
            uwspec(), ubspec(), uwspec(), ubspec(),
        ],
        out_specs=pl.BlockSpec((1, S, H), lambda b, rcs, rus, dds: (b, 0, 0)),
        scratch_shapes=[pltpu.VMEM((S, H), jnp.float32)],
    )

    out = pl.pallas_call(
        _expert_kernel,
        grid_spec=grid_spec,
        out_shape=jax.ShapeDtypeStruct((B, S, H), jnp.float32),
    )(rc, ru, dd, x, mask2,
      cWq, cbq.reshape(EC, 1, H), cWk, cbk.reshape(EC, 1, H),
      cWv, cbv.reshape(EC, 1, H), cWo, cbo.reshape(EC, 1, H),
      uWq, ubq.reshape(EU, 1, H), uWk, ubk.reshape(EU, 1, H),
      uWv, ubv.reshape(EU, 1, H), uWo, ubo.reshape(EU, 1, H))
    return out


# speedup vs baseline: 1.8015x; 1.0312x over previous
import functools

import jax
import jax.numpy as jnp
from jax.experimental import pallas as pl
from jax.experimental.pallas import tpu as pltpu

NH = 12
CAP_FRAC = 1.25


def _transpose_col(v, eye):
    return jax.lax.dot_general(v, eye, (((0,), (0,)), ((), ())),
                               preferred_element_type=jnp.float32,
                               precision=jax.lax.Precision.HIGHEST)


def _routing_kernel(x_ref, wsc_ref, bsc_ref, wsu_ref, bsu_ref,
                    rc_ref, ru_ref, dd_ref, acc_ref, *, cap):
    b = pl.program_id(0)
    nb = pl.num_programs(0)
    seq = x_ref.shape[1]
    xm = jnp.sum(x_ref[0], axis=0, keepdims=True) * (1.0 / seq)
    acc_ref[pl.ds(b, 1), :] = xm

    @pl.when(b == nb - 1)
    def _finalize():
        xall = acc_ref[...]
        bsz = xall.shape[0]
        eye = (jax.lax.broadcasted_iota(jnp.int32, (bsz, bsz), 0) ==
               jax.lax.broadcasted_iota(jnp.int32, (bsz, bsz), 1)
               ).astype(jnp.float32)

        def route(w_ref, b_ref):
            logits = jnp.dot(xall, w_ref[...],
                             preferred_element_type=jnp.float32,
                             precision=jax.lax.Precision.HIGHEST) + b_ref[...]
            p = jax.nn.softmax(logits, axis=-1)
            pmax = jnp.max(p, axis=-1, keepdims=True)
            ne = logits.shape[1]
            col = jax.lax.broadcasted_iota(jnp.int32, logits.shape, 1)
            r = jnp.min(jnp.where(p >= pmax, col, ne), axis=-1,
                        keepdims=True)
            return r.astype(jnp.float32), pmax

        rc, pmc = route(wsc_ref, bsc_ref)
        ru, _ = route(wsu_ref, bsu_ref)

        rc_row = _transpose_col(rc, eye)
        pm_row = _transpose_col(pmc, eye)
        idx_col = jax.lax.broadcasted_iota(jnp.int32, (bsz, bsz), 0)
        idx_row = jax.lax.broadcasted_iota(jnp.int32, (bsz, bsz), 1)
        same = rc_row == rc
        beats = (pm_row > pmc) | ((pm_row == pmc) & (idx_row < idx_col))
        rank = jnp.sum(jnp.where(same & beats, 1.0, 0.0), axis=-1,
                       keepdims=True)
        dropped = jnp.where(rank >= cap, 1.0, 0.0)

        rc_ref[...] = rc_row.astype(jnp.int32)
        ru_ref[...] = _transpose_col(ru, eye).astype(jnp.int32)
        dd_ref[...] = _transpose_col(dropped, eye).astype(jnp.int32)


def _expert_kernel(rc_ref, ru_ref, dd_ref,
                   x_ref, mask_ref,
                   cwq, cbq, cwk, cbk, cwv, cbv, cwo, cbo,
                   uwq, ubq, uwk, ubk, uwv, ubv, uwo, ubo,
                   o_ref, ctx_ref):
    b = pl.program_id(0)
    x = x_ref[0]
    seq, hid = x.shape
    dh = hid // NH
    scale = 1.0 / (dh ** 0.5)
    ext = (1.0 - mask_ref[0].astype(jnp.float32)) * -10000.0

    def mha(wq, bq, wk, bk, wv, bv, wo, bo):
        q = jnp.dot(x, wq[0], preferred_element_type=jnp.float32) + bq[0]
        k = jnp.dot(x, wk[0], preferred_element_type=jnp.float32) + bk[0]
        v = jnp.dot(x, wv[0], preferred_element_type=jnp.float32) + bv[0]
        for h in range(NH):
            sl = slice(h * dh, (h + 1) * dh)
            qh, kh, vh = q[:, sl], k[:, sl], v[:, sl]
            s = jax.lax.dot_general(qh, kh, (((1,), (1,)), ((), ())),
                                    preferred_element_type=jnp.float32)
            s = s * scale + ext
            s = s - jnp.max(s, axis=-1, keepdims=True)
            e = jnp.exp(s)
            p = e / jnp.sum(e, axis=-1, keepdims=True)
            ctx_ref[:, sl] = jnp.dot(p, vh, preferred_element_type=jnp.float32)
        return jnp.dot(ctx_ref[...], wo[0],
                       preferred_element_type=jnp.float32) + bo[0]

    o_ref[0] = mha(cwq, cbq, cwk, cbk, cwv, cbv, cwo, cbo)

    @pl.when(dd_ref[b] == 1)
    def _tail():
        o_ref[0] = o_ref[0] + mha(uwq, ubq, uwk, ubk, uwv, ubv, uwo, ubo)


def kernel(hidden_states, attention_mask, Wsc, bsc, Wsu, bsu,
           cWq, cbq, cWk, cbk, cWv, cbv, cWo, cbo,
           uWq, ubq, uWk, ubk, uWv, ubv, uWo, ubo):
    x = hidden_states
    B, S, H = x.shape
    EC = Wsc.shape[1]
    EU = Wsu.shape[1]
    cap = int(CAP_FRAC * B / EC)

    rc, ru, dd = pl.pallas_call(
        functools.partial(_routing_kernel, cap=cap),
        grid=(B,),
        in_specs=[
            pl.BlockSpec((1, S, H), lambda b: (b, 0, 0)),
            pl.BlockSpec((H, EC), lambda b: (0, 0)),
            pl.BlockSpec((1, EC), lambda b: (0, 0)),
            pl.BlockSpec((H, EU), lambda b: (0, 0)),
            pl.BlockSpec((1, EU), lambda b: (0, 0)),
        ],
        out_specs=[
            pl.BlockSpec((1, B), lambda b: (0, 0)),
            pl.BlockSpec((1, B), lambda b: (0, 0)),
            pl.BlockSpec((1, B), lambda b: (0, 0)),
        ],
        out_shape=[
            jax.ShapeDtypeStruct((1, B), jnp.int32),
            jax.ShapeDtypeStruct((1, B), jnp.int32),
            jax.ShapeDtypeStruct((1, B), jnp.int32),
        ],
        scratch_shapes=[pltpu.VMEM((B, H), jnp.float32)],
    )(x, Wsc, bsc.reshape(1, EC), Wsu, bsu.reshape(1, EU))
    rc = rc.reshape(B)
    ru = ru.reshape(B)
    dd = dd.reshape(B)

    mask2 = attention_mask.reshape(B, 1, S)

    def wspec():
        return pl.BlockSpec((1, H, H),
                            lambda b, rcs, rus, dds: (rcs[b], 0, 0))

    def bspec():
        return pl.BlockSpec((1, 1, H),
                            lambda b, rcs, rus, dds: (rcs[b], 0, 0))

    def uwspec():
        return pl.BlockSpec(
            (1, H, H),
            lambda b, rcs, rus, dds: (jnp.where(dds[b] == 1, rus[b], 0), 0, 0))

    def ubspec():
        return pl.BlockSpec(
            (1, 1, H),
            lambda b, rcs, rus, dds: (jnp.where(dds[b] == 1, rus[b], 0), 0, 0))

    grid_spec = pltpu.PrefetchScalarGridSpec(
        num_scalar_prefetch=3,
        grid=(B,),
        in_specs=[
            pl.BlockSpec((1, S, H), lambda b, rcs, rus, dds: (b, 0, 0)),
            pl.BlockSpec((1, 1, S), lambda b, rcs, rus, dds: (b, 0, 0)),
            wspec(), bspec(), wspec(), bspec(),
            wspec(), bspec(), wspec(), bspec(),
            uwspec(), ubspec(), uwspec(), ubspec(),
            uwspec(), ubspec(), uwspec(), ubspec(),
        ],
        out_specs=pl.BlockSpec((1, S, H), lambda b, rcs, rus, dds: (b, 0, 0)),
        scratch_shapes=[pltpu.VMEM((S, H), jnp.float32)],
    )

    out = pl.pallas_call(
        _expert_kernel,
        grid_spec=grid_spec,
        out_shape=jax.ShapeDtypeStruct((B, S, H), jnp.float32),
    )(rc, ru, dd, x, mask2,
      cWq, cbq.reshape(EC, 1, H), cWk, cbk.reshape(EC, 1, H),
      cWv, cbv.reshape(EC, 1, H), cWo, cbo.reshape(EC, 1, H),
      uWq, ubq.reshape(EU, 1, H), uWk, ubk.reshape(EU, 1, H),
      uWv, ubv.reshape(EU, 1, H), uWo, ubo.reshape(EU, 1, H))
    return out
